# Initial kernel scaffold; baseline (speedup 1.0000x reference)
import jax
import jax.numpy as jnp
from jax.experimental import pallas as pl


def kernel(outp_ctxt, input_ids, lengths):
    raise NotImplementedError("write your pallas kernel here")


# speedup vs baseline: 1.8868x; 1.8868x over previous
import functools

import jax
import jax.numpy as jnp
from jax import lax
from jax.experimental import pallas as pl
from jax.experimental.pallas import tpu as pltpu
from jax.experimental.pallas import tpu_sc as plsc

_NUM_PAD = 2
_B, _L, _D = 8, 2048, 1024
_OUT_MAX = _B * (_L - _NUM_PAD) + _NUM_PAD
_OUT_PAD = _B * _L
_NW = 32
_RPW = _OUT_PAD // _NW
_C = 32
_NCH = _RPW // _C
_DUMP = _OUT_PAD - 1


def _sc_body(ctxt_hbm, ids_hbm, len_hbm, ones_hbm,
             out_ctxt, out_ids, out_mask, out_len,
             len_v, ids_v, ones_v, rows_v, src_idx, dst_idx,
             ids_out_v, mask_v, sem):
  wid = lax.axis_index("s") * 2 + lax.axis_index("c")
  base = wid * _RPW

  pltpu.sync_copy(len_hbm, len_v)
  pltpu.sync_copy(ids_hbm, ids_v)
  pltpu.sync_copy(ones_hbm, ones_v)

  lane = lax.iota(jnp.int32, 16)
  lengths = len_v[...]
  inner = jnp.where(lane < _B, lengths - _NUM_PAD, 0)
  csum = jnp.cumsum(inner)
  offsets = csum - inner + 1

  def lane_val(vec, k):
    return jnp.max(jnp.where(lane == k, vec, 0))

  pf = 1 + jnp.max(csum)
  len0 = lane_val(lengths, 0)
  o = [lane_val(offsets, k) for k in range(1, _B)]
  term = [_L - (o[0] - 1)] + [_L - (o[k] - o[k - 1]) for k in range(1, _B - 1)]

  def src_for(r):
    s = r
    for k in range(_B - 1):
      s = s + jnp.where(r >= o[k], term[k], 0)
    s = jnp.where(r == pf, len0 - 1, s)
    return jnp.minimum(s, _OUT_PAD - 1)

  nrows = jnp.clip(pf + 1 - base, 0, _RPW)
  nch = (nrows + _C - 1) // _C

  def chunk_pack(c, carry):
    cb = base + c * _C
    for g in range(_C // 16):
      r = cb + g * 16 + lane
      src_idx[pl.ds(g * 16, 16)] = src_for(r)
      dst_idx[pl.ds(g * 16, 16)] = jnp.where(r <= pf, r, _DUMP)
    pltpu.async_copy(ctxt_hbm.at[src_idx], rows_v, sem).wait()
    pltpu.async_copy(rows_v, out_ctxt.at[dst_idx], sem).wait()
    return carry

  lax.fori_loop(0, nch, chunk_pack, 0)

  s1 = jnp.clip(pf + 1 - base, 0, _RPW) // _C

  def chunk_ones(c, carry):
    cb = base + c * _C
    for g in range(_C // 16):
      r = cb + g * 16 + lane
      dst_idx[pl.ds(g * 16, 16)] = jnp.where(r > pf, r, _DUMP)
    pltpu.async_copy(ones_v, out_ctxt.at[dst_idx], sem).wait()
    return carry

  lax.fori_loop(s1, _NCH, chunk_ones, 0)

  for g in range(_RPW // 16):
    r = base + g * 16 + lane
    val = plsc.load_gather(ids_v, [src_for(r)])
    ids_out_v[pl.ds(g * 16, 16)] = jnp.where(r <= pf, val, 1)
  pltpu.sync_copy(ids_out_v, out_ids.at[pl.ds(base, _RPW)])

  sent_len = lane_val(lengths, base >> 11)
  for g in range(_RPW // 16):
    p = base + g * 16 + lane
    mask_v[pl.ds(g * 16, 16)] = (
        (p & (_L - 1)) < sent_len).astype(jnp.int32)
  pltpu.sync_copy(mask_v, out_mask.at[pl.ds(base, _RPW)])

  @pl.when(wid == 0)
  def _():
    pltpu.sync_copy(len_v, out_len)


@jax.jit
def kernel(outp_ctxt, input_ids, lengths):
  ctxt_flat = outp_ctxt.reshape(_B * _L, _D)
  ids_flat = input_ids.reshape(_B * _L)
  len_pad = jnp.zeros((16,), jnp.int32).at[:_B].set(lengths)
  ones_buf = jnp.ones((_C, _D), jnp.float32)

  fn = pl.kernel(
      _sc_body,
      out_type=(
          jax.ShapeDtypeStruct((_OUT_PAD, _D), jnp.float32),
          jax.ShapeDtypeStruct((_OUT_PAD,), jnp.int32),
          jax.ShapeDtypeStruct((_OUT_PAD,), jnp.int32),
          jax.ShapeDtypeStruct((16,), jnp.int32),
      ),
      mesh=plsc.VectorSubcoreMesh(core_axis_name="c", subcore_axis_name="s"),
      compiler_params=pltpu.CompilerParams(needs_layout_passes=False),
      scratch_types=[
          pltpu.VMEM((16,), jnp.int32),
          pltpu.VMEM((_OUT_PAD,), jnp.int32),
          pltpu.VMEM((_C, _D), jnp.float32),
          pltpu.VMEM((_C, _D), jnp.float32),
          pltpu.VMEM((_C,), jnp.int32),
          pltpu.VMEM((_C,), jnp.int32),
          pltpu.VMEM((_RPW,), jnp.int32),
          pltpu.VMEM((_RPW,), jnp.int32),
          pltpu.SemaphoreType.DMA,
      ],
  )
  out_ctxt, out_ids, out_mask, out_len = fn(
      ctxt_flat, ids_flat, len_pad, ones_buf)

  out_temp = out_ctxt[:_OUT_MAX][None]
  input_ids_made = out_ids[:_OUT_MAX][None]
  ctxt_mask = out_mask.reshape(_B, _L)
  length = out_len[:_B]
  return (out_temp, input_ids_made, outp_ctxt, ctxt_mask, length)


# speedup vs baseline: 1.9349x; 1.0255x over previous
import functools

import jax
import jax.numpy as jnp
from jax import lax
from jax.experimental import pallas as pl
from jax.experimental.pallas import tpu as pltpu
from jax.experimental.pallas import tpu_sc as plsc

_NUM_PAD = 2
_B, _L, _D = 8, 2048, 1024
_OUT_MAX = _B * (_L - _NUM_PAD) + _NUM_PAD
_OUT_PAD = _B * _L
_NW = 32
_RPW = _OUT_PAD // _NW
_C = 32
_NCH = _RPW // _C
_NPAIR = _NCH // 2
_DUMP = _OUT_PAD - 1


def _sc_body(ctxt_hbm, ids_hbm, len_hbm, ones_hbm,
             out_ctxt, out_ids, out_mask, out_len,
             len_v, ids_v, rows_a, rows_b,
             src_a, src_b, dst_a, dst_b, dst_oa, dst_ob,
             ids_out_v, mask_v,
             sem_ga, sem_gb, sem_sa, sem_sb, sem_oa, sem_ob):
  wid = lax.axis_index("s") * 2 + lax.axis_index("c")
  base = wid * _RPW

  pltpu.sync_copy(len_hbm, len_v)
  pltpu.sync_copy(ids_hbm, ids_v)
  pltpu.sync_copy(ones_hbm, rows_a)

  lane = lax.iota(jnp.int32, 16)
  lengths = len_v[...]
  inner = jnp.where(lane < _B, lengths - _NUM_PAD, 0)
  csum = jnp.cumsum(inner)
  offsets = csum - inner + 1

  def lane_val(vec, k):
    return jnp.max(jnp.where(lane == k, vec, 0))

  pf = 1 + jnp.max(csum)
  len0 = lane_val(lengths, 0)
  o = [lane_val(offsets, k) for k in range(1, _B)]
  term = [_L - (o[0] - 1)] + [_L - (o[k] - o[k - 1]) for k in range(1, _B - 1)]

  def src_for(r):
    s = r
    for k in range(_B - 1):
      s = s + jnp.where(r >= o[k], term[k], 0)
    s = jnp.where(r == pf, len0 - 1, s)
    return jnp.minimum(s, _OUT_PAD - 1)

  nrows = jnp.clip(pf + 1 - base, 0, _RPW)
  nch = (nrows + _C - 1) // _C
  s1 = nrows // _C

  def fill_ones_dst(c, dst):
    cb = base + c * _C
    for g in range(_C // 16):
      r = cb + g * 16 + lane
      dst[pl.ds(g * 16, 16)] = jnp.where(r > pf, r, _DUMP)

  p0 = s1 // 2

  def ones_pair(i, carry):
    @pl.when(i > p0)
    def _():
      pltpu.make_async_copy(rows_a, out_ctxt.at[dst_oa], sem_oa).wait()
      pltpu.make_async_copy(rows_a, out_ctxt.at[dst_ob], sem_ob).wait()
    fill_ones_dst(2 * i, dst_oa)
    pltpu.async_copy(rows_a, out_ctxt.at[dst_oa], sem_oa)
    fill_ones_dst(2 * i + 1, dst_ob)
    pltpu.async_copy(rows_a, out_ctxt.at[dst_ob], sem_ob)
    return carry

  lax.fori_loop(p0, _NPAIR, ones_pair, 0)

  @pl.when(p0 < _NPAIR)
  def _():
    pltpu.make_async_copy(rows_a, out_ctxt.at[dst_oa], sem_oa).wait()
    pltpu.make_async_copy(rows_a, out_ctxt.at[dst_ob], sem_ob).wait()

  def fill_pack_idx(c, src, dst):
    cb = base + c * _C
    for g in range(_C // 16):
      r = cb + g * 16 + lane
      src[pl.ds(g * 16, 16)] = src_for(r)
      dst[pl.ds(g * 16, 16)] = jnp.where(r <= pf, r, _DUMP)

  npair = (nch + 1) // 2

  def pack_pair(i, carry):
    @pl.when(i > 0)
    def _():
      pltpu.make_async_copy(rows_a, out_ctxt.at[dst_a], sem_sa).wait()
      pltpu.make_async_copy(rows_b, out_ctxt.at[dst_b], sem_sb).wait()
    fill_pack_idx(2 * i, src_a, dst_a)
    pltpu.async_copy(ctxt_hbm.at[src_a], rows_a, sem_ga)
    fill_pack_idx(2 * i + 1, src_b, dst_b)
    pltpu.async_copy(ctxt_hbm.at[src_b], rows_b, sem_gb)
    pltpu.make_async_copy(ctxt_hbm.at[src_a], rows_a, sem_ga).wait()
    pltpu.async_copy(rows_a, out_ctxt.at[dst_a], sem_sa)
    pltpu.make_async_copy(ctxt_hbm.at[src_b], rows_b, sem_gb).wait()
    pltpu.async_copy(rows_b, out_ctxt.at[dst_b], sem_sb)
    return carry

  lax.fori_loop(0, npair, pack_pair, 0)

  @pl.when(npair > 0)
  def _():
    pltpu.make_async_copy(rows_a, out_ctxt.at[dst_a], sem_sa).wait()
    pltpu.make_async_copy(rows_b, out_ctxt.at[dst_b], sem_sb).wait()

  for g in range(_RPW // 16):
    r = base + g * 16 + lane
    val = plsc.load_gather(ids_v, [src_for(r)])
    ids_out_v[pl.ds(g * 16, 16)] = jnp.where(r <= pf, val, 1)
  pltpu.sync_copy(ids_out_v, out_ids.at[pl.ds(base, _RPW)])

  sent_len = lane_val(lengths, base >> 11)
  for g in range(_RPW // 16):
    p = base + g * 16 + lane
    mask_v[pl.ds(g * 16, 16)] = (
        (p & (_L - 1)) < sent_len).astype(jnp.int32)
  pltpu.sync_copy(mask_v, out_mask.at[pl.ds(base, _RPW)])

  @pl.when(wid == 0)
  def _():
    pltpu.sync_copy(len_v, out_len)


@jax.jit
def kernel(outp_ctxt, input_ids, lengths):
  ctxt_flat = outp_ctxt.reshape(_B * _L, _D)
  ids_flat = input_ids.reshape(_B * _L)
  len_pad = jnp.zeros((16,), jnp.int32).at[:_B].set(lengths)
  ones_buf = jnp.ones((_C, _D), jnp.float32)

  fn = pl.kernel(
      _sc_body,
      out_type=(
          jax.ShapeDtypeStruct((_OUT_PAD, _D), jnp.float32),
          jax.ShapeDtypeStruct((_OUT_PAD,), jnp.int32),
          jax.ShapeDtypeStruct((_OUT_PAD,), jnp.int32),
          jax.ShapeDtypeStruct((16,), jnp.int32),
      ),
      mesh=plsc.VectorSubcoreMesh(core_axis_name="c", subcore_axis_name="s"),
      compiler_params=pltpu.CompilerParams(needs_layout_passes=False),
      scratch_types=[
          pltpu.VMEM((16,), jnp.int32),
          pltpu.VMEM((_OUT_PAD,), jnp.int32),
          pltpu.VMEM((_C, _D), jnp.float32),
          pltpu.VMEM((_C, _D), jnp.float32),
          pltpu.VMEM((_C,), jnp.int32),
          pltpu.VMEM((_C,), jnp.int32),
          pltpu.VMEM((_C,), jnp.int32),
          pltpu.VMEM((_C,), jnp.int32),
          pltpu.VMEM((_C,), jnp.int32),
          pltpu.VMEM((_C,), jnp.int32),
          pltpu.VMEM((_RPW,), jnp.int32),
          pltpu.VMEM((_RPW,), jnp.int32),
          pltpu.SemaphoreType.DMA,
          pltpu.SemaphoreType.DMA,
          pltpu.SemaphoreType.DMA,
          pltpu.SemaphoreType.DMA,
          pltpu.SemaphoreType.DMA,
          pltpu.SemaphoreType.DMA,
      ],
  )
  out_ctxt, out_ids, out_mask, out_len = fn(
      ctxt_flat, ids_flat, len_pad, ones_buf)

  out_temp = out_ctxt[:_OUT_MAX][None]
  input_ids_made = out_ids[:_OUT_MAX][None]
  ctxt_mask = out_mask.reshape(_B, _L)
  length = out_len[:_B]
  return (out_temp, input_ids_made, outp_ctxt, ctxt_mask, length)


# speedup vs baseline: 2.3117x; 1.1947x over previous
import functools

import jax
import jax.numpy as jnp
from jax import lax
from jax.experimental import pallas as pl
from jax.experimental.pallas import tpu as pltpu
from jax.experimental.pallas import tpu_sc as plsc

_NUM_PAD = 2
_B, _L, _D = 8, 2048, 1024
_OUT_MAX = _B * (_L - _NUM_PAD) + _NUM_PAD
_OUT_PAD = _B * _L
_NW = 32
_RPW = _OUT_PAD // _NW
_C = 32
_NCH = _RPW // _C
_NPAIR = _NCH // 2


def _sc_body(ctxt_hbm, ids_hbm, len_hbm, ones_hbm,
             out_ctxt, out_ids, out_mask, out_len,
             len_v, ids_v, rows_a, rows_b, rows_c,
             src_a, src_b, dst_a, dst_b, dst_oa, dst_ob, src_c, dst_c,
             ids_out_v, mask_v,
             sem_ga, sem_gb, sem_sa, sem_sb, sem_oa, sem_ob):
  wid = lax.axis_index("s") * 2 + lax.axis_index("c")
  base = wid * _RPW
  wlast = jnp.minimum(base + _RPW - 1, _OUT_MAX - 1)

  pltpu.sync_copy(len_hbm, len_v)
  pltpu.sync_copy(ids_hbm, ids_v)
  pltpu.sync_copy(ones_hbm, rows_a)

  lane = lax.iota(jnp.int32, 16)
  lengths = len_v[...]
  inner = jnp.where(lane < _B, lengths - _NUM_PAD, 0)
  csum = jnp.cumsum(inner)
  offsets = csum - inner + 1

  def lane_val(vec, k):
    return jnp.max(jnp.where(lane == k, vec, 0))

  pf = 1 + jnp.max(csum)
  len0 = lane_val(lengths, 0)
  o = [lane_val(offsets, k) for k in range(1, _B)]
  term = [_L - (o[0] - 1)] + [_L - (o[k] - o[k - 1]) for k in range(1, _B - 1)]

  def src_for(r):
    s = r
    for k in range(_B - 1):
      s = s + jnp.where(r >= o[k], term[k], 0)
    s = jnp.where(r == pf, len0 - 1, s)
    return jnp.minimum(s, _OUT_PAD - 1)

  nrows = jnp.clip(pf + 1 - base, 0, _RPW)
  nch = (nrows + _C - 1) // _C
  s1 = nrows // _C

  def fill_ones_dst(c, dst):
    cb = base + c * _C
    for g in range(_C // 16):
      r = cb + g * 16 + lane
      dst[pl.ds(g * 16, 16)] = jnp.where((r > pf) & (r <= wlast), r, wlast)

  p0 = s1 // 2

  def ones_pair(i, carry):
    @pl.when(i > p0)
    def _():
      pltpu.make_async_copy(rows_a, out_ctxt.at[dst_oa], sem_oa).wait()
      pltpu.make_async_copy(rows_a, out_ctxt.at[dst_ob], sem_ob).wait()
    fill_ones_dst(2 * i, dst_oa)
    pltpu.async_copy(rows_a, out_ctxt.at[dst_oa], sem_oa)
    fill_ones_dst(2 * i + 1, dst_ob)
    pltpu.async_copy(rows_a, out_ctxt.at[dst_ob], sem_ob)
    return carry

  @pl.when(pf < wlast)
  def _():
    lax.fori_loop(p0, _NPAIR, ones_pair, 0)
    pltpu.make_async_copy(rows_a, out_ctxt.at[dst_oa], sem_oa).wait()
    pltpu.make_async_copy(rows_a, out_ctxt.at[dst_ob], sem_ob).wait()

  def fill_pack_idx(c, src, dst):
    cb = base + c * _C
    for g in range(_C // 16):
      r = cb + g * 16 + lane
      src[pl.ds(g * 16, 16)] = src_for(r)
      dst[pl.ds(g * 16, 16)] = jnp.where(r <= pf, r, base)

  npair = (nch + 1) // 2

  def pack_pair(i, carry):
    @pl.when(i > 0)
    def _():
      pltpu.make_async_copy(rows_a, out_ctxt.at[dst_a], sem_sa).wait()
      pltpu.make_async_copy(rows_b, out_ctxt.at[dst_b], sem_sb).wait()
    fill_pack_idx(2 * i, src_a, dst_a)
    pltpu.async_copy(ctxt_hbm.at[src_a], rows_a, sem_ga)
    fill_pack_idx(2 * i + 1, src_b, dst_b)
    pltpu.async_copy(ctxt_hbm.at[src_b], rows_b, sem_gb)
    pltpu.make_async_copy(ctxt_hbm.at[src_a], rows_a, sem_ga).wait()
    pltpu.async_copy(rows_a, out_ctxt.at[dst_a], sem_sa)
    pltpu.make_async_copy(ctxt_hbm.at[src_b], rows_b, sem_gb).wait()
    pltpu.async_copy(rows_b, out_ctxt.at[dst_b], sem_sb)
    return carry

  @pl.when(npair > 0)
  def _():
    lax.fori_loop(0, npair, pack_pair, 0)
    pltpu.make_async_copy(rows_a, out_ctxt.at[dst_a], sem_sa).wait()
    pltpu.make_async_copy(rows_b, out_ctxt.at[dst_b], sem_sb).wait()
    src_c[...] = jnp.broadcast_to(src_for(base), (16,))
    dst_c[...] = jnp.broadcast_to(base, (16,))
    pltpu.async_copy(ctxt_hbm.at[src_c], rows_c, sem_ga).wait()
    pltpu.async_copy(rows_c, out_ctxt.at[dst_c], sem_sa).wait()

  for g in range(_RPW // 16):
    r = base + g * 16 + lane
    val = plsc.load_gather(ids_v, [src_for(r)])
    ids_out_v[pl.ds(g * 16, 16)] = jnp.where(r <= pf, val, 1)
  pltpu.sync_copy(ids_out_v, out_ids.at[pl.ds(base, _RPW)])

  sent_len = lane_val(lengths, base >> 11)
  for g in range(_RPW // 16):
    p = base + g * 16 + lane
    mask_v[pl.ds(g * 16, 16)] = (
        (p & (_L - 1)) < sent_len).astype(jnp.int32)
  pltpu.sync_copy(mask_v, out_mask.at[pl.ds(base, _RPW)])

  @pl.when(wid == 0)
  def _():
    pltpu.sync_copy(len_v, out_len)


@jax.jit
def kernel(outp_ctxt, input_ids, lengths):
  ctxt_flat = outp_ctxt.reshape(_B * _L, _D)
  ids_flat = input_ids.reshape(_B * _L)
  len_pad = jnp.zeros((16,), jnp.int32).at[:_B].set(lengths)
  ones_buf = jnp.ones((_C, _D), jnp.float32)

  fn = pl.kernel(
      _sc_body,
      out_type=(
          jax.ShapeDtypeStruct((_OUT_MAX, _D), jnp.float32),
          jax.ShapeDtypeStruct((_OUT_PAD,), jnp.int32),
          jax.ShapeDtypeStruct((_OUT_PAD,), jnp.int32),
          jax.ShapeDtypeStruct((16,), jnp.int32),
      ),
      mesh=plsc.VectorSubcoreMesh(core_axis_name="c", subcore_axis_name="s"),
      compiler_params=pltpu.CompilerParams(needs_layout_passes=False),
      scratch_types=[
          pltpu.VMEM((16,), jnp.int32),
          pltpu.VMEM((_OUT_PAD,), jnp.int32),
          pltpu.VMEM((_C, _D), jnp.float32),
          pltpu.VMEM((_C, _D), jnp.float32),
          pltpu.VMEM((16, _D), jnp.float32),
          pltpu.VMEM((_C,), jnp.int32),
          pltpu.VMEM((_C,), jnp.int32),
          pltpu.VMEM((_C,), jnp.int32),
          pltpu.VMEM((_C,), jnp.int32),
          pltpu.VMEM((_C,), jnp.int32),
          pltpu.VMEM((_C,), jnp.int32),
          pltpu.VMEM((16,), jnp.int32),
          pltpu.VMEM((16,), jnp.int32),
          pltpu.VMEM((_RPW,), jnp.int32),
          pltpu.VMEM((_RPW,), jnp.int32),
          pltpu.SemaphoreType.DMA,
          pltpu.SemaphoreType.DMA,
          pltpu.SemaphoreType.DMA,
          pltpu.SemaphoreType.DMA,
          pltpu.SemaphoreType.DMA,
          pltpu.SemaphoreType.DMA,
      ],
  )
  out_ctxt, out_ids, out_mask, out_len = fn(
      ctxt_flat, ids_flat, len_pad, ones_buf)

  out_temp = out_ctxt[None]
  input_ids_made = out_ids[:_OUT_MAX][None]
  ctxt_mask = out_mask.reshape(_B, _L)
  length = out_len[:_B]
  return (out_temp, input_ids_made, outp_ctxt, ctxt_mask, length)
